# baseline (speedup 1.0000x reference)
import functools

import jax
import jax.numpy as jnp
from jax import lax
from jax.experimental import pallas as pl
from jax.experimental.pallas import tpu as pltpu
from jax.experimental.pallas import tpu_sc as plsc

NC, NS = 2, 16
NW = NC * NS
CHUNK = 128
SUPER = 40


def _mm_body(x_ref, w_ref, o_ref):
    o_ref[...] = jnp.dot(x_ref[...], w_ref[...],
                         preferred_element_type=jnp.float32)


def _relu_mm_body(p_ref, w_ref, o_ref):
    h = jnp.maximum(p_ref[0] + p_ref[1], 0.0)
    o_ref[...] = jnp.dot(h, w_ref[...], preferred_element_type=jnp.float32)


def _readout_body(q_ref, o_ref):
    i = pl.program_id(0)
    h = jnp.maximum(q_ref[0] + q_ref[1], 0.0)
    s = jnp.sum(h, axis=0, keepdims=True)

    @pl.when(i == 0)
    def _():
        o_ref[...] = s

    @pl.when(i > 0)
    def _():
        o_ref[...] += s


def _tc_matmul(x, w, block_rows):
    n, d = x.shape
    grid = n // block_rows
    return pl.pallas_call(
        _mm_body,
        grid=(grid,),
        in_specs=[
            pl.BlockSpec((block_rows, d), lambda i: (i, 0)),
            pl.BlockSpec((d, w.shape[1]), lambda i: (0, 0)),
        ],
        out_specs=pl.BlockSpec((block_rows, w.shape[1]), lambda i: (i, 0)),
        out_shape=jax.ShapeDtypeStruct((n, w.shape[1]), jnp.float32),
    )(x, w)


def _tc_relu_matmul(p, w, block_rows):
    _, n, d = p.shape
    grid = n // block_rows
    return pl.pallas_call(
        _relu_mm_body,
        grid=(grid,),
        in_specs=[
            pl.BlockSpec((2, block_rows, d), lambda i: (0, i, 0)),
            pl.BlockSpec((d, w.shape[1]), lambda i: (0, 0)),
        ],
        out_specs=pl.BlockSpec((block_rows, w.shape[1]), lambda i: (i, 0)),
        out_shape=jax.ShapeDtypeStruct((n, w.shape[1]), jnp.float32),
    )(p, w)


def _tc_readout(q, block_rows):
    _, n, d = q.shape
    grid = n // block_rows
    out = pl.pallas_call(
        _readout_body,
        grid=(grid,),
        in_specs=[pl.BlockSpec((2, block_rows, d), lambda i: (0, i, 0))],
        out_specs=pl.BlockSpec((1, d), lambda i: (0, 0)),
        out_shape=jax.ShapeDtypeStruct((1, d), jnp.float32),
    )(q)
    return out.reshape(d)


def _make_sc_agg(npad, d, nsuper):
    rows_pt = npad // NS
    npairs = SUPER // 2

    mesh = plsc.VectorSubcoreMesh(
        core_axis_name="c", subcore_axis_name="s",
        num_cores=NC, num_subcores=NS)

    @functools.partial(
        pl.kernel,
        out_type=jax.ShapeDtypeStruct((NC, npad, d), jnp.float32),
        mesh=mesh,
        scratch_types=[
            pltpu.VMEM((2, SUPER, CHUNK), jnp.int32),
            pltpu.VMEM((CHUNK, d), jnp.float32),
            pltpu.VMEM((CHUNK, d), jnp.float32),
            pltpu.VMEM_SHARED((npad, d), jnp.float32),
            pltpu.SemaphoreType.DMA,
            pltpu.SemaphoreType.DMA,
            pltpu.SemaphoreType.DMA,
            pltpu.SemaphoreType.DMA,
        ],
    )
    def agg(y_hbm, idx_hbm, zero_hbm, out_hbm,
            idx_v, rows_a, rows_b, acc, sem_a, sem_b, ssem_a, ssem_b):
        c = lax.axis_index("c")
        s = lax.axis_index("s")
        wid = s * NC + c
        row0 = pl.multiple_of(s * rows_pt, 8)

        pltpu.sync_copy(zero_hbm.at[pl.ds(row0, rows_pt)],
                        acc.at[pl.ds(row0, rows_pt)])
        plsc.subcore_barrier()

        def gather(i, rows, sem):
            pltpu.async_copy(y_hbm.at[idx_v.at[0, i]], rows, sem)

        def wait_g(rows, sem):
            pltpu.make_async_copy(y_hbm.at[idx_v.at[0, 0]], rows, sem).wait()

        def scatter(i, rows, ssem):
            pltpu.async_copy(rows, acc.at[idx_v.at[1, i]], ssem, add=True)

        def wait_s(rows, ssem):
            pltpu.make_async_copy(rows, acc.at[idx_v.at[1, 0]], ssem).wait()

        def superchunk(sc, carry):
            pltpu.sync_copy(idx_hbm.at[wid, sc], idx_v)
            gather(0, rows_a, sem_a)
            gather(1, rows_b, sem_b)

            def pair(j, carry2):
                i0 = j * 2
                wait_g(rows_a, sem_a)
                scatter(i0, rows_a, ssem_a)
                wait_g(rows_b, sem_b)
                scatter(i0 + 1, rows_b, ssem_b)
                wait_s(rows_a, ssem_a)
                gather(i0 + 2, rows_a, sem_a)
                wait_s(rows_b, ssem_b)
                gather(i0 + 3, rows_b, sem_b)
                return carry2

            lax.fori_loop(0, npairs - 1, pair, 0)
            wait_g(rows_a, sem_a)
            scatter(SUPER - 2, rows_a, ssem_a)
            wait_g(rows_b, sem_b)
            scatter(SUPER - 1, rows_b, ssem_b)
            wait_s(rows_a, ssem_a)
            wait_s(rows_b, ssem_b)
            return carry

        lax.fori_loop(0, nsuper, superchunk, 0)
        plsc.subcore_barrier()

        pltpu.sync_copy(acc.at[pl.ds(row0, rows_pt)],
                        out_hbm.at[c, pl.ds(row0, rows_pt)])

    return agg


def kernel(x, edge_index, W1, W2):
    n, d = x.shape
    e = edge_index.shape[1]

    npad = ((n + NS * 8 - 1) // (NS * 8)) * (NS * 8)
    if npad == n:
        npad += NS * 8
    grp = NW * SUPER * CHUNK
    epad = ((e + grp - 1) // grp) * grp
    nsuper = epad // grp
    pad_idx = n + jnp.arange(epad - e, dtype=jnp.int32) % (npad - n)
    src = jnp.concatenate([edge_index[0], pad_idx])
    dst = jnp.concatenate([edge_index[1], pad_idx])
    idx = jnp.stack([src.reshape(NW, nsuper, SUPER, CHUNK),
                     dst.reshape(NW, nsuper, SUPER, CHUNK)], axis=2)

    xp = jnp.zeros((npad, d), jnp.float32).at[:n].set(x)
    zeros = jnp.zeros((npad, d), jnp.float32)

    block_rows = npad // NS
    agg = _make_sc_agg(npad, d, nsuper)

    y1 = _tc_matmul(xp, W1, block_rows)
    p1 = agg(y1, idx, zeros)
    y2 = _tc_relu_matmul(p1, W2, block_rows)
    p2 = agg(y2, idx, zeros)
    return _tc_readout(p2, block_rows)


# speedup vs baseline: 1.3344x; 1.3344x over previous
import functools

import jax
import jax.numpy as jnp
from jax import lax
from jax.experimental import pallas as pl
from jax.experimental.pallas import tpu as pltpu
from jax.experimental.pallas import tpu_sc as plsc

NC, NS = 2, 16
NW = NC * NS
CHUNK = 128
SUPER = 40


def _mm_body(x_ref, w_ref, o_ref):
    o_ref[...] = jnp.dot(x_ref[...], w_ref[...],
                         preferred_element_type=jnp.float32)


def _relu_mm_body(p_ref, w_ref, o_ref):
    h = jnp.maximum(p_ref[0] + p_ref[1], 0.0)
    o_ref[...] = jnp.dot(h, w_ref[...], preferred_element_type=jnp.float32)


def _readout_body(q_ref, o_ref):
    i = pl.program_id(0)
    h = jnp.maximum(q_ref[0] + q_ref[1], 0.0)
    s = jnp.sum(h, axis=0, keepdims=True)

    @pl.when(i == 0)
    def _():
        o_ref[...] = s

    @pl.when(i > 0)
    def _():
        o_ref[...] += s


def _tc_matmul(x, w, block_rows):
    n, d = x.shape
    grid = n // block_rows
    return pl.pallas_call(
        _mm_body,
        grid=(grid,),
        in_specs=[
            pl.BlockSpec((block_rows, d), lambda i: (i, 0)),
            pl.BlockSpec((d, w.shape[1]), lambda i: (0, 0)),
        ],
        out_specs=pl.BlockSpec((block_rows, w.shape[1]), lambda i: (i, 0)),
        out_shape=jax.ShapeDtypeStruct((n, w.shape[1]), jnp.float32),
    )(x, w)


def _tc_relu_matmul(p, w, block_rows):
    _, n, d = p.shape
    grid = n // block_rows
    return pl.pallas_call(
        _relu_mm_body,
        grid=(grid,),
        in_specs=[
            pl.BlockSpec((2, block_rows, d), lambda i: (0, i, 0)),
            pl.BlockSpec((d, w.shape[1]), lambda i: (0, 0)),
        ],
        out_specs=pl.BlockSpec((block_rows, w.shape[1]), lambda i: (i, 0)),
        out_shape=jax.ShapeDtypeStruct((n, w.shape[1]), jnp.float32),
    )(p, w)


def _tc_readout(q, block_rows):
    _, n, d = q.shape
    grid = n // block_rows
    out = pl.pallas_call(
        _readout_body,
        grid=(grid,),
        in_specs=[pl.BlockSpec((2, block_rows, d), lambda i: (0, i, 0))],
        out_specs=pl.BlockSpec((1, d), lambda i: (0, 0)),
        out_shape=jax.ShapeDtypeStruct((1, d), jnp.float32),
    )(q)
    return out.reshape(d)


def _make_sc_agg(npad, d, nsuper):
    rows_pt = npad // NS
    npairs = SUPER // 2

    mesh = plsc.VectorSubcoreMesh(
        core_axis_name="c", subcore_axis_name="s",
        num_cores=NC, num_subcores=NS)

    @functools.partial(
        pl.kernel,
        out_type=jax.ShapeDtypeStruct((NC, npad, d), jnp.float32),
        mesh=mesh,
        scratch_types=[
            pltpu.VMEM((2, SUPER, CHUNK), jnp.int32),
            pltpu.VMEM((CHUNK, d), jnp.float32),
            pltpu.VMEM((CHUNK, d), jnp.float32),
            pltpu.VMEM_SHARED((npad, d), jnp.float32),
            pltpu.SemaphoreType.DMA,
            pltpu.SemaphoreType.DMA,
        ],
    )
    def agg(y_hbm, idx_hbm, zero_hbm, out_hbm,
            idx_v, rows_a, rows_b, acc, sem_a, sem_b):
        c = lax.axis_index("c")
        s = lax.axis_index("s")
        wid = s * NC + c
        row0 = pl.multiple_of(s * rows_pt, 8)

        pltpu.sync_copy(zero_hbm.at[pl.ds(row0, rows_pt)],
                        acc.at[pl.ds(row0, rows_pt)])
        plsc.subcore_barrier()

        def gather(i, rows, sem):
            pltpu.async_copy(y_hbm.at[idx_v.at[0, i]], rows, sem)

        def wait(rows, sem):
            pltpu.make_async_copy(y_hbm.at[idx_v.at[0, 0]], rows, sem).wait()

        def scatter(i, rows):
            pltpu.sync_copy(rows, acc.at[idx_v.at[1, i]], add=True)

        def superchunk(sc, carry):
            pltpu.sync_copy(idx_hbm.at[wid, sc], idx_v)
            gather(0, rows_a, sem_a)
            gather(1, rows_b, sem_b)

            def pair(j, carry2):
                i0 = j * 2
                wait(rows_a, sem_a)
                scatter(i0, rows_a)
                gather(i0 + 2, rows_a, sem_a)
                wait(rows_b, sem_b)
                scatter(i0 + 1, rows_b)
                gather(i0 + 3, rows_b, sem_b)
                return carry2

            lax.fori_loop(0, npairs - 1, pair, 0)
            wait(rows_a, sem_a)
            scatter(SUPER - 2, rows_a)
            wait(rows_b, sem_b)
            scatter(SUPER - 1, rows_b)
            return carry

        lax.fori_loop(0, nsuper, superchunk, 0)
        plsc.subcore_barrier()

        pltpu.sync_copy(acc.at[pl.ds(row0, rows_pt)],
                        out_hbm.at[c, pl.ds(row0, rows_pt)])

    return agg


def kernel(x, edge_index, W1, W2):
    n, d = x.shape
    e = edge_index.shape[1]

    npad = ((n + NS * 8 - 1) // (NS * 8)) * (NS * 8)
    if npad == n:
        npad += NS * 8
    grp = NW * SUPER * CHUNK
    epad = ((e + grp - 1) // grp) * grp
    nsuper = epad // grp
    pad_idx = n + jnp.arange(epad - e, dtype=jnp.int32) % (npad - n)
    src = jnp.concatenate([edge_index[0], pad_idx])
    dst = jnp.concatenate([edge_index[1], pad_idx])
    idx = jnp.stack([src.reshape(NW, nsuper, SUPER, CHUNK),
                     dst.reshape(NW, nsuper, SUPER, CHUNK)], axis=2)

    xp = jnp.zeros((npad, d), jnp.float32).at[:n].set(x)
    zeros = jnp.zeros((npad, d), jnp.float32)

    block_rows = npad // 4
    agg = _make_sc_agg(npad, d, nsuper)

    y1 = _tc_matmul(xp, W1, block_rows)
    p1 = agg(y1, idx, zeros)
    y2 = _tc_relu_matmul(p1, W2, block_rows)
    p2 = agg(y2, idx, zeros)
    return _tc_readout(p2, block_rows)


# speedup vs baseline: 1.3396x; 1.0039x over previous
import functools

import jax
import jax.numpy as jnp
from jax import lax
from jax.experimental import pallas as pl
from jax.experimental.pallas import tpu as pltpu
from jax.experimental.pallas import tpu_sc as plsc

NC, NS = 2, 16
NW = NC * NS
CHUNK = 128
SUPER = 40


def _mm_body(x_ref, w_ref, o_ref):
    o_ref[...] = jnp.dot(x_ref[...], w_ref[...],
                         preferred_element_type=jnp.float32)


def _relu_mm_body(p_ref, w_ref, o_ref):
    h = jnp.maximum(p_ref[0] + p_ref[1], 0.0)
    o_ref[...] = jnp.dot(h, w_ref[...], preferred_element_type=jnp.float32)


def _readout_body(q_ref, o_ref):
    i = pl.program_id(0)
    h = jnp.maximum(q_ref[0] + q_ref[1], 0.0)
    s = jnp.sum(h, axis=0, keepdims=True)

    @pl.when(i == 0)
    def _():
        o_ref[...] = s

    @pl.when(i > 0)
    def _():
        o_ref[...] += s


def _tc_matmul(x, w, block_rows):
    n, d = x.shape
    grid = n // block_rows
    return pl.pallas_call(
        _mm_body,
        grid=(grid,),
        in_specs=[
            pl.BlockSpec((block_rows, d), lambda i: (i, 0)),
            pl.BlockSpec((d, w.shape[1]), lambda i: (0, 0)),
        ],
        out_specs=pl.BlockSpec((block_rows, w.shape[1]), lambda i: (i, 0)),
        out_shape=jax.ShapeDtypeStruct((n, w.shape[1]), jnp.float32),
    )(x, w)


def _tc_relu_matmul(p, w, block_rows):
    _, n, d = p.shape
    grid = n // block_rows
    return pl.pallas_call(
        _relu_mm_body,
        grid=(grid,),
        in_specs=[
            pl.BlockSpec((2, block_rows, d), lambda i: (0, i, 0)),
            pl.BlockSpec((d, w.shape[1]), lambda i: (0, 0)),
        ],
        out_specs=pl.BlockSpec((block_rows, w.shape[1]), lambda i: (i, 0)),
        out_shape=jax.ShapeDtypeStruct((n, w.shape[1]), jnp.float32),
    )(p, w)


def _tc_readout(q, block_rows):
    _, n, d = q.shape
    grid = n // block_rows
    out = pl.pallas_call(
        _readout_body,
        grid=(grid,),
        in_specs=[pl.BlockSpec((2, block_rows, d), lambda i: (0, i, 0))],
        out_specs=pl.BlockSpec((1, d), lambda i: (0, 0)),
        out_shape=jax.ShapeDtypeStruct((1, d), jnp.float32),
    )(q)
    return out.reshape(d)


def _make_sc_agg(npad, d, nsuper):
    rows_pt = npad // NS
    npairs = SUPER // 2

    mesh = plsc.VectorSubcoreMesh(
        core_axis_name="c", subcore_axis_name="s",
        num_cores=NC, num_subcores=NS)

    @functools.partial(
        pl.kernel,
        out_type=jax.ShapeDtypeStruct((NC, npad, d), jnp.float32),
        mesh=mesh,
        scratch_types=[
            pltpu.VMEM((2, SUPER, CHUNK), jnp.int32),
            pltpu.VMEM((CHUNK, d), jnp.float32),
            pltpu.VMEM((CHUNK, d), jnp.float32),
            pltpu.VMEM_SHARED((npad, d), jnp.float32),
            pltpu.SemaphoreType.DMA,
            pltpu.SemaphoreType.DMA,
        ],
    )
    def agg(y_hbm, idx_hbm, zero_hbm, out_hbm,
            idx_v, rows_a, rows_b, acc, sem_a, sem_b):
        c = lax.axis_index("c")
        s = lax.axis_index("s")
        wid = s * NC + c
        row0 = pl.multiple_of(s * rows_pt, 8)

        pltpu.sync_copy(zero_hbm.at[pl.ds(row0, rows_pt)],
                        acc.at[pl.ds(row0, rows_pt)])
        plsc.subcore_barrier()

        def gather(i, rows, sem):
            pltpu.async_copy(y_hbm.at[idx_v.at[0, i]], rows, sem)

        def wait(rows, sem):
            pltpu.make_async_copy(y_hbm.at[idx_v.at[0, 0]], rows, sem).wait()

        def scatter(i, rows):
            pltpu.sync_copy(rows, acc.at[idx_v.at[1, i]], add=True)

        def superchunk(sc, carry):
            pltpu.sync_copy(idx_hbm.at[wid, sc], idx_v)
            gather(0, rows_a, sem_a)
            gather(1, rows_b, sem_b)

            def pair(j, carry2):
                i0 = j * 2
                wait(rows_a, sem_a)
                scatter(i0, rows_a)
                gather(i0 + 2, rows_a, sem_a)
                wait(rows_b, sem_b)
                scatter(i0 + 1, rows_b)
                gather(i0 + 3, rows_b, sem_b)
                return carry2

            lax.fori_loop(0, npairs - 1, pair, 0)
            wait(rows_a, sem_a)
            scatter(SUPER - 2, rows_a)
            wait(rows_b, sem_b)
            scatter(SUPER - 1, rows_b)
            return carry

        lax.fori_loop(0, nsuper, superchunk, 0)
        plsc.subcore_barrier()

        pltpu.sync_copy(acc.at[pl.ds(row0, rows_pt)],
                        out_hbm.at[c, pl.ds(row0, rows_pt)])

    return agg


def kernel(x, edge_index, W1, W2):
    n, d = x.shape
    e = edge_index.shape[1]

    npad = ((n + NS * 8 - 1) // (NS * 8)) * (NS * 8)
    if npad == n:
        npad += NS * 8
    grp = NW * SUPER * CHUNK
    epad = ((e + grp - 1) // grp) * grp
    nsuper = epad // grp
    pad_idx = n + jnp.arange(epad - e, dtype=jnp.int32) % (npad - n)
    src = jnp.concatenate([edge_index[0], pad_idx])
    dst = jnp.concatenate([edge_index[1], pad_idx])
    idx = jnp.stack([src.reshape(NW, nsuper, SUPER, CHUNK),
                     dst.reshape(NW, nsuper, SUPER, CHUNK)], axis=2)

    xp = jnp.zeros((npad, d), jnp.float32).at[:n].set(x)
    zeros = jnp.zeros((npad, d), jnp.float32)

    block_rows = npad
    agg = _make_sc_agg(npad, d, nsuper)

    y1 = _tc_matmul(xp, W1, block_rows)
    p1 = agg(y1, idx, zeros)
    y2 = _tc_relu_matmul(p1, W2, block_rows)
    p2 = agg(y2, idx, zeros)
    return _tc_readout(p2, block_rows)
